# baseline (speedup 1.0000x reference)
import functools

import jax
import jax.numpy as jnp
from jax.experimental import pallas as pl
from jax.experimental.pallas import tpu as pltpu
from jax.experimental.pallas import tpu_sc as plsc


def _sc_gather4(user_ids, item_ids, gmf_user, gmf_item, mlp_user, mlp_item):
    B = user_ids.shape[0]
    D = gmf_user.shape[1]
    info = plsc.get_sparse_core_info()
    nw = info.num_cores * info.num_subcores
    assert B % (8 * nw) == 0
    b_per_w = B // nw
    mesh = plsc.VectorSubcoreMesh(core_axis_name="c", subcore_axis_name="s")
    out_t = jax.ShapeDtypeStruct((B, D), jnp.float32)

    @functools.partial(
        pl.kernel,
        mesh=mesh,
        out_type=(out_t, out_t, out_t, out_t),
        scratch_types=[
            pltpu.VMEM((b_per_w,), jnp.int32),
            pltpu.VMEM((b_per_w,), jnp.int32),
            pltpu.VMEM((b_per_w, D), jnp.float32),
            pltpu.VMEM((b_per_w, D), jnp.float32),
            pltpu.SemaphoreType.DMA,
            pltpu.SemaphoreType.DMA,
        ],
        compiler_params=pltpu.CompilerParams(use_tc_tiling_on_sc=False),
    )
    def k(uid_hbm, iid_hbm, t_gu, t_gi, t_mu, t_mi,
          o_gu, o_gi, o_mu, o_mi, idx_u, idx_i, rows_a, rows_b, sem_a, sem_b):
        wid = jax.lax.axis_index("s") * info.num_cores + jax.lax.axis_index("c")
        base = wid * b_per_w
        sl = pl.ds(base, b_per_w)
        pltpu.sync_copy(uid_hbm.at[sl], idx_u)
        pltpu.sync_copy(iid_hbm.at[sl], idx_i)
        cp_a = pltpu.async_copy(t_gu.at[idx_u], rows_a, sem_a)
        cp_b = pltpu.async_copy(t_gi.at[idx_i], rows_b, sem_b)
        cp_a.wait()
        pltpu.sync_copy(rows_a, o_gu.at[sl])
        cp_a = pltpu.async_copy(t_mu.at[idx_u], rows_a, sem_a)
        cp_b.wait()
        pltpu.sync_copy(rows_b, o_gi.at[sl])
        cp_b = pltpu.async_copy(t_mi.at[idx_i], rows_b, sem_b)
        cp_a.wait()
        pltpu.sync_copy(rows_a, o_mu.at[sl])
        cp_b.wait()
        pltpu.sync_copy(rows_b, o_mi.at[sl])

    return k(user_ids, item_ids, gmf_user, gmf_item, mlp_user, mlp_item)


def _tc_body(gu, gi, mu, mi, w1a, w1b, b1, w2, b2, w3, b3, w4, b4,
             wg, wx, bout, out):
    h = jnp.maximum(
        jnp.dot(mu[...], w1a[...], preferred_element_type=jnp.float32)
        + jnp.dot(mi[...], w1b[...], preferred_element_type=jnp.float32)
        + b1[...], 0.0)
    h = jnp.maximum(
        jnp.dot(h, w2[...], preferred_element_type=jnp.float32) + b2[...], 0.0)
    h = jnp.maximum(
        jnp.dot(h, w3[...], preferred_element_type=jnp.float32) + b3[...], 0.0)
    h = jnp.maximum(
        jnp.dot(h, w4[...], preferred_element_type=jnp.float32) + b4[...], 0.0)
    gmf = gu[...] * gi[...]
    pred = (jnp.sum(gmf * wg[...], axis=1)
            + jnp.sum(h * wx[...], axis=1) + bout[0, 0])
    out[...] = jax.nn.sigmoid(pred)


def _tc_mlp(gu, gi, mu, mi, W1, b1, W2, b2, W3, b3, W4, b4, Wout, bout):
    B, D = gu.shape
    BB = 2048
    grid = (B // BB,)
    d1 = W1.shape[1]
    row = lambda m, n: pl.BlockSpec((m, n), lambda i: (0, 0))
    blk = lambda n: pl.BlockSpec((BB, n), lambda i: (i, 0))
    return pl.pallas_call(
        _tc_body,
        grid=grid,
        in_specs=[
            blk(D), blk(D), blk(D), blk(D),
            row(D, d1), row(D, d1), row(1, d1),
            row(d1, W2.shape[1]), row(1, W2.shape[1]),
            row(W3.shape[0], W3.shape[1]), row(1, W3.shape[1]),
            row(W4.shape[0], W4.shape[1]), row(1, W4.shape[1]),
            row(1, D), row(1, W4.shape[1]), row(1, 1),
        ],
        out_specs=pl.BlockSpec((BB,), lambda i: (i,)),
        out_shape=jax.ShapeDtypeStruct((B,), jnp.float32),
        compiler_params=pltpu.CompilerParams(
            dimension_semantics=("parallel",)),
    )(gu, gi, mu, mi,
      W1[:D], W1[D:], b1.reshape(1, d1),
      W2, b2.reshape(1, -1), W3, b3.reshape(1, -1), W4, b4.reshape(1, -1),
      Wout[:D].reshape(1, D), Wout[D:].reshape(1, -1), bout.reshape(1, 1))


def kernel(user_ids, item_ids, gmf_user, gmf_item, mlp_user, mlp_item,
           W1, b1, W2, b2, W3, b3, W4, b4, Wout, bout):
    gu, gi, mu, mi = _sc_gather4(user_ids, item_ids,
                                 gmf_user, gmf_item, mlp_user, mlp_item)
    return _tc_mlp(gu, gi, mu, mi, W1, b1, W2, b2, W3, b3, W4, b4, Wout, bout)


# speedup vs baseline: 1.2148x; 1.2148x over previous
import functools

import jax
import jax.numpy as jnp
from jax.experimental import pallas as pl
from jax.experimental.pallas import tpu as pltpu
from jax.experimental.pallas import tpu_sc as plsc


def _sc_gather2(user_ids, item_ids, tab_u, tab_i):
    B = user_ids.shape[0]
    W = tab_u.shape[1]
    info = plsc.get_sparse_core_info()
    nw = info.num_cores * info.num_subcores
    assert B % (8 * nw) == 0
    b_per_w = B // nw
    C = 256
    n_chunks = b_per_w // C
    mesh = plsc.VectorSubcoreMesh(core_axis_name="c", subcore_axis_name="s")
    out_t = jax.ShapeDtypeStruct((B, W), jnp.float32)

    @functools.partial(
        pl.kernel,
        mesh=mesh,
        out_type=(out_t, out_t),
        scratch_types=[
            pltpu.VMEM((C,), jnp.int32),
            pltpu.VMEM((C,), jnp.int32),
            pltpu.VMEM((C, W), jnp.float32),
            pltpu.VMEM((C, W), jnp.float32),
            pltpu.SemaphoreType.DMA,
            pltpu.SemaphoreType.DMA,
        ],
    )
    def k(uid_hbm, iid_hbm, tu, ti, o_u, o_i,
          idx_u, idx_i, rows_u, rows_i, sem_u, sem_i):
        wid = jax.lax.axis_index("s") * info.num_cores + jax.lax.axis_index("c")
        for c in range(n_chunks):
            base = wid * b_per_w + c * C
            sl = pl.ds(base, C)
            pltpu.sync_copy(uid_hbm.at[sl], idx_u)
            pltpu.sync_copy(iid_hbm.at[sl], idx_i)
            cp_u = pltpu.async_copy(tu.at[idx_u], rows_u, sem_u)
            cp_i = pltpu.async_copy(ti.at[idx_i], rows_i, sem_i)
            cp_u.wait()
            pltpu.sync_copy(rows_u, o_u.at[sl])
            cp_i.wait()
            pltpu.sync_copy(rows_i, o_i.at[sl])

    return k(user_ids, item_ids, tab_u, tab_i)


def _tc_body(u, i, p, q, b1, w2, b2, w3, b3, w4, b4, wg, wx, bout, out):
    uv = u[...]
    iv = i[...]
    h = jnp.maximum(
        jnp.dot(uv, p[...], preferred_element_type=jnp.float32)
        + jnp.dot(iv, q[...], preferred_element_type=jnp.float32)
        + b1[...], 0.0)
    h = jnp.maximum(
        jnp.dot(h, w2[...], preferred_element_type=jnp.float32) + b2[...], 0.0)
    h = jnp.maximum(
        jnp.dot(h, w3[...], preferred_element_type=jnp.float32) + b3[...], 0.0)
    h = jnp.maximum(
        jnp.dot(h, w4[...], preferred_element_type=jnp.float32) + b4[...], 0.0)
    pred = (jnp.sum(uv * iv * wg[...], axis=1)
            + jnp.sum(h * wx[...], axis=1) + bout[0, 0])
    out[...] = jax.nn.sigmoid(pred)


def _tc_mlp(u_rows, i_rows, W1, b1, W2, b2, W3, b3, W4, b4, Wout, bout):
    B, W = u_rows.shape
    D = W // 2
    BB = 2048
    grid = (B // BB,)
    d1 = W1.shape[1]
    zpad = jnp.zeros((D, d1), jnp.float32)
    p = jnp.concatenate([zpad, W1[:D]], axis=0)
    q = jnp.concatenate([zpad, W1[D:]], axis=0)
    wg = jnp.concatenate([Wout[:D, 0], jnp.zeros((D,), jnp.float32)])
    row = lambda m, n: pl.BlockSpec((m, n), lambda idx: (0, 0))
    blk = lambda n: pl.BlockSpec((BB, n), lambda idx: (idx, 0))
    return pl.pallas_call(
        _tc_body,
        grid=grid,
        in_specs=[
            blk(W), blk(W),
            row(W, d1), row(W, d1), row(1, d1),
            row(d1, W2.shape[1]), row(1, W2.shape[1]),
            row(W3.shape[0], W3.shape[1]), row(1, W3.shape[1]),
            row(W4.shape[0], W4.shape[1]), row(1, W4.shape[1]),
            row(1, W), row(1, W4.shape[1]), row(1, 1),
        ],
        out_specs=pl.BlockSpec((BB,), lambda idx: (idx,)),
        out_shape=jax.ShapeDtypeStruct((B,), jnp.float32),
        compiler_params=pltpu.CompilerParams(
            dimension_semantics=("parallel",)),
    )(u_rows, i_rows,
      p, q, b1.reshape(1, d1),
      W2, b2.reshape(1, -1), W3, b3.reshape(1, -1), W4, b4.reshape(1, -1),
      wg.reshape(1, W), Wout[D:].reshape(1, -1), bout.reshape(1, 1))


def kernel(user_ids, item_ids, gmf_user, gmf_item, mlp_user, mlp_item,
           W1, b1, W2, b2, W3, b3, W4, b4, Wout, bout):
    tab_u = jnp.concatenate([gmf_user, mlp_user], axis=1)
    tab_i = jnp.concatenate([gmf_item, mlp_item], axis=1)
    u_rows, i_rows = _sc_gather2(user_ids, item_ids, tab_u, tab_i)
    return _tc_mlp(u_rows, i_rows, W1, b1, W2, b2, W3, b3, W4, b4, Wout, bout)


# speedup vs baseline: 1.6340x; 1.3450x over previous
import functools

import jax
import jax.numpy as jnp
from jax.experimental import pallas as pl
from jax.experimental.pallas import tpu as pltpu
from jax.experimental.pallas import tpu_sc as plsc


def _build_body(g, m, p1, p2, out):
    out[...] = (
        jax.lax.dot_general(g[...], p1[...], (((0,), (0,)), ((), ())),
                            preferred_element_type=jnp.float32)
        + jax.lax.dot_general(m[...], p2[...], (((0,), (0,)), ((), ())),
                              preferred_element_type=jnp.float32))


def _build_table(g_t, m_t):
    D, V = g_t.shape
    BT = 2048
    grid = (pl.cdiv(V, BT),)
    p1 = jnp.concatenate(
        [jnp.eye(D, dtype=jnp.float32),
         jnp.zeros((D, D), jnp.float32)], axis=1)
    p2 = jnp.concatenate(
        [jnp.zeros((D, D), jnp.float32),
         jnp.eye(D, dtype=jnp.float32)], axis=1)
    return pl.pallas_call(
        _build_body,
        grid=grid,
        in_specs=[
            pl.BlockSpec((D, BT), lambda i: (0, i)),
            pl.BlockSpec((D, BT), lambda i: (0, i)),
            pl.BlockSpec((D, 2 * D), lambda i: (0, 0)),
            pl.BlockSpec((D, 2 * D), lambda i: (0, 0)),
        ],
        out_specs=pl.BlockSpec((BT, 2 * D), lambda i: (i, 0)),
        out_shape=jax.ShapeDtypeStruct((V, 2 * D), jnp.float32),
        compiler_params=pltpu.CompilerParams(
            dimension_semantics=("arbitrary",)),
    )(g_t, m_t, p1, p2)


def _sc_gather2(user_ids, item_ids, tab_u, tab_i):
    B = user_ids.shape[0]
    W = tab_u.shape[1]
    info = plsc.get_sparse_core_info()
    nw = info.num_cores * info.num_subcores
    assert B % (8 * nw) == 0
    b_per_w = B // nw
    C = 256
    n_chunks = b_per_w // C
    mesh = plsc.VectorSubcoreMesh(core_axis_name="c", subcore_axis_name="s")
    out_t = jax.ShapeDtypeStruct((B, W), jnp.float32)

    @functools.partial(
        pl.kernel,
        mesh=mesh,
        out_type=(out_t, out_t),
        scratch_types=[
            pltpu.VMEM((C,), jnp.int32),
            pltpu.VMEM((C,), jnp.int32),
            pltpu.VMEM((C, W), jnp.float32),
            pltpu.VMEM((C, W), jnp.float32),
            pltpu.SemaphoreType.DMA,
            pltpu.SemaphoreType.DMA,
        ],
    )
    def k(uid_hbm, iid_hbm, tu, ti, o_u, o_i,
          idx_u, idx_i, rows_u, rows_i, sem_u, sem_i):
        wid = jax.lax.axis_index("s") * info.num_cores + jax.lax.axis_index("c")
        for c in range(n_chunks):
            base = wid * b_per_w + c * C
            sl = pl.ds(base, C)
            pltpu.sync_copy(uid_hbm.at[sl], idx_u)
            pltpu.sync_copy(iid_hbm.at[sl], idx_i)
            cp_u = pltpu.async_copy(tu.at[idx_u], rows_u, sem_u)
            cp_i = pltpu.async_copy(ti.at[idx_i], rows_i, sem_i)
            cp_u.wait()
            pltpu.sync_copy(rows_u, o_u.at[sl])
            cp_i.wait()
            pltpu.sync_copy(rows_i, o_i.at[sl])

    return k(user_ids, item_ids, tab_u, tab_i)


def _tc_body(u, i, p, q, b1, w2, b2, w3, b3, w4, b4, wg, wx, bout, out):
    uv = u[...]
    iv = i[...]
    h = jnp.maximum(
        jnp.dot(uv, p[...], preferred_element_type=jnp.float32)
        + jnp.dot(iv, q[...], preferred_element_type=jnp.float32)
        + b1[...], 0.0)
    h = jnp.maximum(
        jnp.dot(h, w2[...], preferred_element_type=jnp.float32) + b2[...], 0.0)
    h = jnp.maximum(
        jnp.dot(h, w3[...], preferred_element_type=jnp.float32) + b3[...], 0.0)
    h = jnp.maximum(
        jnp.dot(h, w4[...], preferred_element_type=jnp.float32) + b4[...], 0.0)
    pred = (jnp.sum(uv * iv * wg[...], axis=1)
            + jnp.sum(h * wx[...], axis=1) + bout[0, 0])
    out[...] = jax.nn.sigmoid(pred)


def _tc_mlp(u_rows, i_rows, W1, b1, W2, b2, W3, b3, W4, b4, Wout, bout):
    B, W = u_rows.shape
    D = W // 2
    BB = 2048
    grid = (B // BB,)
    d1 = W1.shape[1]
    zpad = jnp.zeros((D, d1), jnp.float32)
    p = jnp.concatenate([zpad, W1[:D]], axis=0)
    q = jnp.concatenate([zpad, W1[D:]], axis=0)
    wg = jnp.concatenate([Wout[:D, 0], jnp.zeros((D,), jnp.float32)])
    row = lambda m, n: pl.BlockSpec((m, n), lambda idx: (0, 0))
    blk = lambda n: pl.BlockSpec((BB, n), lambda idx: (idx, 0))
    return pl.pallas_call(
        _tc_body,
        grid=grid,
        in_specs=[
            blk(W), blk(W),
            row(W, d1), row(W, d1), row(1, d1),
            row(d1, W2.shape[1]), row(1, W2.shape[1]),
            row(W3.shape[0], W3.shape[1]), row(1, W3.shape[1]),
            row(W4.shape[0], W4.shape[1]), row(1, W4.shape[1]),
            row(1, W), row(1, W4.shape[1]), row(1, 1),
        ],
        out_specs=pl.BlockSpec((BB,), lambda idx: (idx,)),
        out_shape=jax.ShapeDtypeStruct((B,), jnp.float32),
        compiler_params=pltpu.CompilerParams(
            dimension_semantics=("parallel",)),
    )(u_rows, i_rows,
      p, q, b1.reshape(1, d1),
      W2, b2.reshape(1, -1), W3, b3.reshape(1, -1), W4, b4.reshape(1, -1),
      wg.reshape(1, W), Wout[D:].reshape(1, -1), bout.reshape(1, 1))


def kernel(user_ids, item_ids, gmf_user, gmf_item, mlp_user, mlp_item,
           W1, b1, W2, b2, W3, b3, W4, b4, Wout, bout):
    tab_u = _build_table(gmf_user.T, mlp_user.T)
    tab_i = _build_table(gmf_item.T, mlp_item.T)
    u_rows, i_rows = _sc_gather2(user_ids, item_ids, tab_u, tab_i)
    return _tc_mlp(u_rows, i_rows, W1, b1, W2, b2, W3, b3, W4, b4, Wout, bout)


# speedup vs baseline: 1.8485x; 1.1313x over previous
import functools

import jax
import jax.numpy as jnp
from jax.experimental import pallas as pl
from jax.experimental.pallas import tpu as pltpu
from jax.experimental.pallas import tpu_sc as plsc


def _build_body(gu, mu, gi, mi, out_u, out_i):
    out_u[...] = jnp.concatenate(
        [jnp.swapaxes(gu[...], 0, 1), jnp.swapaxes(mu[...], 0, 1)], axis=1)
    out_i[...] = jnp.concatenate(
        [jnp.swapaxes(gi[...], 0, 1), jnp.swapaxes(mi[...], 0, 1)], axis=1)


def _build_tables(gu_t, mu_t, gi_t, mi_t):
    D, V = gu_t.shape
    BT = 2048
    grid = (pl.cdiv(V, BT),)
    inspec = pl.BlockSpec((D, BT), lambda i: (0, i))
    out_t = jax.ShapeDtypeStruct((V, 2 * D), jnp.float32)
    return pl.pallas_call(
        _build_body,
        grid=grid,
        in_specs=[inspec, inspec, inspec, inspec],
        out_specs=(pl.BlockSpec((BT, 2 * D), lambda i: (i, 0)),
                   pl.BlockSpec((BT, 2 * D), lambda i: (i, 0))),
        out_shape=(out_t, out_t),
        compiler_params=pltpu.CompilerParams(
            dimension_semantics=("arbitrary",)),
    )(gu_t, mu_t, gi_t, mi_t)


def _sc_gather2(user_ids, item_ids, tab_u, tab_i):
    B = user_ids.shape[0]
    W = tab_u.shape[1]
    info = plsc.get_sparse_core_info()
    nw = info.num_cores * info.num_subcores
    assert B % (8 * nw) == 0
    b_per_w = B // nw
    C = 256
    n_chunks = b_per_w // C
    mesh = plsc.VectorSubcoreMesh(core_axis_name="c", subcore_axis_name="s")
    out_t = jax.ShapeDtypeStruct((B, W), jnp.float32)

    @functools.partial(
        pl.kernel,
        mesh=mesh,
        out_type=(out_t, out_t),
        scratch_types=[
            pltpu.VMEM((C,), jnp.int32),
            pltpu.VMEM((C,), jnp.int32),
            pltpu.VMEM((C, W), jnp.float32),
            pltpu.VMEM((C, W), jnp.float32),
            pltpu.SemaphoreType.DMA,
            pltpu.SemaphoreType.DMA,
        ],
    )
    def k(uid_hbm, iid_hbm, tu, ti, o_u, o_i,
          idx_u, idx_i, rows_u, rows_i, sem_u, sem_i):
        wid = jax.lax.axis_index("s") * info.num_cores + jax.lax.axis_index("c")
        for c in range(n_chunks):
            base = wid * b_per_w + c * C
            sl = pl.ds(base, C)
            pltpu.sync_copy(uid_hbm.at[sl], idx_u)
            pltpu.sync_copy(iid_hbm.at[sl], idx_i)
            cp_u = pltpu.async_copy(tu.at[idx_u], rows_u, sem_u)
            cp_i = pltpu.async_copy(ti.at[idx_i], rows_i, sem_i)
            cp_u.wait()
            pltpu.sync_copy(rows_u, o_u.at[sl])
            cp_i.wait()
            pltpu.sync_copy(rows_i, o_i.at[sl])

    return k(user_ids, item_ids, tab_u, tab_i)


def _tc_body(u, i, p, q, b1, w2, b2, w3, b3, w4, b4, wg, wx, bout, out):
    uv = u[...]
    iv = i[...]
    h = jnp.maximum(
        jnp.dot(uv, p[...], preferred_element_type=jnp.float32)
        + jnp.dot(iv, q[...], preferred_element_type=jnp.float32)
        + b1[...], 0.0)
    h = jnp.maximum(
        jnp.dot(h, w2[...], preferred_element_type=jnp.float32) + b2[...], 0.0)
    h = jnp.maximum(
        jnp.dot(h, w3[...], preferred_element_type=jnp.float32) + b3[...], 0.0)
    h = jnp.maximum(
        jnp.dot(h, w4[...], preferred_element_type=jnp.float32) + b4[...], 0.0)
    pred = (jnp.sum(uv * iv * wg[...], axis=1)
            + jnp.sum(h * wx[...], axis=1) + bout[0, 0])
    out[...] = jax.nn.sigmoid(pred)


def _tc_mlp(u_rows, i_rows, W1, b1, W2, b2, W3, b3, W4, b4, Wout, bout):
    B, W = u_rows.shape
    D = W // 2
    BB = 2048
    grid = (B // BB,)
    d1 = W1.shape[1]
    zpad = jnp.zeros((D, d1), jnp.float32)
    p = jnp.concatenate([zpad, W1[:D]], axis=0)
    q = jnp.concatenate([zpad, W1[D:]], axis=0)
    wg = jnp.concatenate([Wout[:D, 0], jnp.zeros((D,), jnp.float32)])
    row = lambda m, n: pl.BlockSpec((m, n), lambda idx: (0, 0))
    blk = lambda n: pl.BlockSpec((BB, n), lambda idx: (idx, 0))
    return pl.pallas_call(
        _tc_body,
        grid=grid,
        in_specs=[
            blk(W), blk(W),
            row(W, d1), row(W, d1), row(1, d1),
            row(d1, W2.shape[1]), row(1, W2.shape[1]),
            row(W3.shape[0], W3.shape[1]), row(1, W3.shape[1]),
            row(W4.shape[0], W4.shape[1]), row(1, W4.shape[1]),
            row(1, W), row(1, W4.shape[1]), row(1, 1),
        ],
        out_specs=pl.BlockSpec((BB,), lambda idx: (idx,)),
        out_shape=jax.ShapeDtypeStruct((B,), jnp.float32),
        compiler_params=pltpu.CompilerParams(
            dimension_semantics=("parallel",)),
    )(u_rows, i_rows,
      p, q, b1.reshape(1, d1),
      W2, b2.reshape(1, -1), W3, b3.reshape(1, -1), W4, b4.reshape(1, -1),
      wg.reshape(1, W), Wout[D:].reshape(1, -1), bout.reshape(1, 1))


def kernel(user_ids, item_ids, gmf_user, gmf_item, mlp_user, mlp_item,
           W1, b1, W2, b2, W3, b3, W4, b4, Wout, bout):
    tab_u, tab_i = _build_tables(gmf_user.T, mlp_user.T,
                                 gmf_item.T, mlp_item.T)
    u_rows, i_rows = _sc_gather2(user_ids, item_ids, tab_u, tab_i)
    return _tc_mlp(u_rows, i_rows, W1, b1, W2, b2, W3, b3, W4, b4, Wout, bout)


# speedup vs baseline: 1.9857x; 1.0742x over previous
import functools

import jax
import jax.numpy as jnp
from jax.experimental import pallas as pl
from jax.experimental.pallas import tpu as pltpu
from jax.experimental.pallas import tpu_sc as plsc


def _build_body(gu, mu, gi, mi, out_u, out_i):
    out_u[...] = jnp.concatenate(
        [jnp.swapaxes(gu[...], 0, 1), jnp.swapaxes(mu[...], 0, 1)], axis=1)
    out_i[...] = jnp.concatenate(
        [jnp.swapaxes(gi[...], 0, 1), jnp.swapaxes(mi[...], 0, 1)], axis=1)


def _build_tables(gu_t, mu_t, gi_t, mi_t):
    D, V = gu_t.shape
    BT = 4096
    grid = (pl.cdiv(V, BT),)
    inspec = pl.BlockSpec((D, BT), lambda i: (0, i))
    out_t = jax.ShapeDtypeStruct((V, 2 * D), jnp.float32)
    return pl.pallas_call(
        _build_body,
        grid=grid,
        in_specs=[inspec, inspec, inspec, inspec],
        out_specs=(pl.BlockSpec((BT, 2 * D), lambda i: (i, 0)),
                   pl.BlockSpec((BT, 2 * D), lambda i: (i, 0))),
        out_shape=(out_t, out_t),
        compiler_params=pltpu.CompilerParams(
            dimension_semantics=("arbitrary",)),
    )(gu_t, mu_t, gi_t, mi_t)


def _sc_gather2(user_ids, item_ids, tab_u, tab_i):
    B = user_ids.shape[0]
    W = tab_u.shape[1]
    info = plsc.get_sparse_core_info()
    nw = info.num_cores * info.num_subcores
    assert B % (8 * nw) == 0
    b_per_w = B // nw
    C = 256
    n_chunks = b_per_w // C
    mesh = plsc.VectorSubcoreMesh(core_axis_name="c", subcore_axis_name="s")
    out_t = jax.ShapeDtypeStruct((B, W), jnp.float32)

    @functools.partial(
        pl.kernel,
        mesh=mesh,
        out_type=(out_t, out_t),
        scratch_types=[
            pltpu.VMEM((C,), jnp.int32),
            pltpu.VMEM((C,), jnp.int32),
            pltpu.VMEM((C, W), jnp.float32),
            pltpu.VMEM((C, W), jnp.float32),
            pltpu.SemaphoreType.DMA,
            pltpu.SemaphoreType.DMA,
        ],
    )
    def k(uid_hbm, iid_hbm, tu, ti, o_u, o_i,
          idx_u, idx_i, rows_u, rows_i, sem_u, sem_i):
        wid = jax.lax.axis_index("s") * info.num_cores + jax.lax.axis_index("c")
        for c in range(n_chunks):
            base = wid * b_per_w + c * C
            sl = pl.ds(base, C)
            pltpu.sync_copy(uid_hbm.at[sl], idx_u)
            pltpu.sync_copy(iid_hbm.at[sl], idx_i)
            cp_u = pltpu.async_copy(tu.at[idx_u], rows_u, sem_u)
            cp_i = pltpu.async_copy(ti.at[idx_i], rows_i, sem_i)
            cp_u.wait()
            pltpu.sync_copy(rows_u, o_u.at[sl])
            cp_i.wait()
            pltpu.sync_copy(rows_i, o_i.at[sl])

    return k(user_ids, item_ids, tab_u, tab_i)


def _tc_body(u, i, p, q, b1, w2, b2, w3, b3, w4, b4, wg, wx, bout, out):
    uv = u[...]
    iv = i[...]
    h = jnp.maximum(
        jnp.dot(uv, p[...], preferred_element_type=jnp.float32)
        + jnp.dot(iv, q[...], preferred_element_type=jnp.float32)
        + b1[...], 0.0)
    h = jnp.maximum(
        jnp.dot(h, w2[...], preferred_element_type=jnp.float32) + b2[...], 0.0)
    h = jnp.maximum(
        jnp.dot(h, w3[...], preferred_element_type=jnp.float32) + b3[...], 0.0)
    h = jnp.maximum(
        jnp.dot(h, w4[...], preferred_element_type=jnp.float32) + b4[...], 0.0)
    pred = (jnp.sum(uv * iv * wg[...], axis=1)
            + jnp.sum(h * wx[...], axis=1) + bout[0, 0])
    out[...] = jax.nn.sigmoid(pred)


def _tc_mlp(u_rows, i_rows, W1, b1, W2, b2, W3, b3, W4, b4, Wout, bout):
    B, W = u_rows.shape
    D = W // 2
    BB = 2048
    grid = (B // BB,)
    d1 = W1.shape[1]
    zpad = jnp.zeros((D, d1), jnp.float32)
    p = jnp.concatenate([zpad, W1[:D]], axis=0)
    q = jnp.concatenate([zpad, W1[D:]], axis=0)
    wg = jnp.concatenate([Wout[:D, 0], jnp.zeros((D,), jnp.float32)])
    row = lambda m, n: pl.BlockSpec((m, n), lambda idx: (0, 0))
    blk = lambda n: pl.BlockSpec((BB, n), lambda idx: (idx, 0))
    return pl.pallas_call(
        _tc_body,
        grid=grid,
        in_specs=[
            blk(W), blk(W),
            row(W, d1), row(W, d1), row(1, d1),
            row(d1, W2.shape[1]), row(1, W2.shape[1]),
            row(W3.shape[0], W3.shape[1]), row(1, W3.shape[1]),
            row(W4.shape[0], W4.shape[1]), row(1, W4.shape[1]),
            row(1, W), row(1, W4.shape[1]), row(1, 1),
        ],
        out_specs=pl.BlockSpec((BB,), lambda idx: (idx,)),
        out_shape=jax.ShapeDtypeStruct((B,), jnp.float32),
        compiler_params=pltpu.CompilerParams(
            dimension_semantics=("parallel",)),
    )(u_rows, i_rows,
      p, q, b1.reshape(1, d1),
      W2, b2.reshape(1, -1), W3, b3.reshape(1, -1), W4, b4.reshape(1, -1),
      wg.reshape(1, W), Wout[D:].reshape(1, -1), bout.reshape(1, 1))


def kernel(user_ids, item_ids, gmf_user, gmf_item, mlp_user, mlp_item,
           W1, b1, W2, b2, W3, b3, W4, b4, Wout, bout):
    tab_u, tab_i = _build_tables(gmf_user.T, mlp_user.T,
                                 gmf_item.T, mlp_item.T)
    u_rows, i_rows = _sc_gather2(user_ids, item_ids, tab_u, tab_i)
    return _tc_mlp(u_rows, i_rows, W1, b1, W2, b2, W3, b3, W4, b4, Wout, bout)


# speedup vs baseline: 2.0128x; 1.0137x over previous
import functools

import jax
import jax.numpy as jnp
from jax.experimental import pallas as pl
from jax.experimental.pallas import tpu as pltpu
from jax.experimental.pallas import tpu_sc as plsc


def _build_body(gu, mu, gi, mi, out_u, out_i):
    out_u[...] = jnp.concatenate(
        [jnp.swapaxes(gu[...], 0, 1), jnp.swapaxes(mu[...], 0, 1)], axis=1)
    out_i[...] = jnp.concatenate(
        [jnp.swapaxes(gi[...], 0, 1), jnp.swapaxes(mi[...], 0, 1)], axis=1)


def _build_tables(gu_t, mu_t, gi_t, mi_t):
    D, V = gu_t.shape
    BT = 8192
    grid = (pl.cdiv(V, BT),)
    inspec = pl.BlockSpec((D, BT), lambda i: (0, i))
    out_t = jax.ShapeDtypeStruct((V, 2 * D), jnp.float32)
    return pl.pallas_call(
        _build_body,
        grid=grid,
        in_specs=[inspec, inspec, inspec, inspec],
        out_specs=(pl.BlockSpec((BT, 2 * D), lambda i: (i, 0)),
                   pl.BlockSpec((BT, 2 * D), lambda i: (i, 0))),
        out_shape=(out_t, out_t),
        compiler_params=pltpu.CompilerParams(
            dimension_semantics=("arbitrary",)),
    )(gu_t, mu_t, gi_t, mi_t)


def _sc_gather2(user_ids, item_ids, tab_u, tab_i):
    B = user_ids.shape[0]
    W = tab_u.shape[1]
    info = plsc.get_sparse_core_info()
    nw = info.num_cores * info.num_subcores
    assert B % (8 * nw) == 0
    b_per_w = B // nw
    C = 256
    n_chunks = b_per_w // C
    mesh = plsc.VectorSubcoreMesh(core_axis_name="c", subcore_axis_name="s")
    out_t = jax.ShapeDtypeStruct((B, W), jnp.float32)

    @functools.partial(
        pl.kernel,
        mesh=mesh,
        out_type=(out_t, out_t),
        scratch_types=[
            pltpu.VMEM((C,), jnp.int32),
            pltpu.VMEM((C,), jnp.int32),
            pltpu.VMEM((C, W), jnp.float32),
            pltpu.VMEM((C, W), jnp.float32),
            pltpu.SemaphoreType.DMA,
            pltpu.SemaphoreType.DMA,
        ],
    )
    def k(uid_hbm, iid_hbm, tu, ti, o_u, o_i,
          idx_u, idx_i, rows_u, rows_i, sem_u, sem_i):
        wid = jax.lax.axis_index("s") * info.num_cores + jax.lax.axis_index("c")
        for c in range(n_chunks):
            base = wid * b_per_w + c * C
            sl = pl.ds(base, C)
            pltpu.sync_copy(uid_hbm.at[sl], idx_u)
            pltpu.sync_copy(iid_hbm.at[sl], idx_i)
            cp_u = pltpu.async_copy(tu.at[idx_u], rows_u, sem_u)
            cp_i = pltpu.async_copy(ti.at[idx_i], rows_i, sem_i)
            cp_u.wait()
            pltpu.sync_copy(rows_u, o_u.at[sl])
            cp_i.wait()
            pltpu.sync_copy(rows_i, o_i.at[sl])

    return k(user_ids, item_ids, tab_u, tab_i)


def _tc_body(u, i, p, q, b1, w2, b2, w3, b3, w4, b4, wg, wx, bout, out):
    uv = u[...]
    iv = i[...]
    h = jnp.maximum(
        jnp.dot(uv, p[...], preferred_element_type=jnp.float32)
        + jnp.dot(iv, q[...], preferred_element_type=jnp.float32)
        + b1[...], 0.0)
    h = jnp.maximum(
        jnp.dot(h, w2[...], preferred_element_type=jnp.float32) + b2[...], 0.0)
    h = jnp.maximum(
        jnp.dot(h, w3[...], preferred_element_type=jnp.float32) + b3[...], 0.0)
    h = jnp.maximum(
        jnp.dot(h, w4[...], preferred_element_type=jnp.float32) + b4[...], 0.0)
    pred = (jnp.sum(uv * iv * wg[...], axis=1)
            + jnp.sum(h * wx[...], axis=1) + bout[0, 0])
    out[...] = jax.nn.sigmoid(pred)


def _tc_mlp(u_rows, i_rows, W1, b1, W2, b2, W3, b3, W4, b4, Wout, bout):
    B, W = u_rows.shape
    D = W // 2
    BB = 2048
    grid = (B // BB,)
    d1 = W1.shape[1]
    zpad = jnp.zeros((D, d1), jnp.float32)
    p = jnp.concatenate([zpad, W1[:D]], axis=0)
    q = jnp.concatenate([zpad, W1[D:]], axis=0)
    wg = jnp.concatenate([Wout[:D, 0], jnp.zeros((D,), jnp.float32)])
    row = lambda m, n: pl.BlockSpec((m, n), lambda idx: (0, 0))
    blk = lambda n: pl.BlockSpec((BB, n), lambda idx: (idx, 0))
    return pl.pallas_call(
        _tc_body,
        grid=grid,
        in_specs=[
            blk(W), blk(W),
            row(W, d1), row(W, d1), row(1, d1),
            row(d1, W2.shape[1]), row(1, W2.shape[1]),
            row(W3.shape[0], W3.shape[1]), row(1, W3.shape[1]),
            row(W4.shape[0], W4.shape[1]), row(1, W4.shape[1]),
            row(1, W), row(1, W4.shape[1]), row(1, 1),
        ],
        out_specs=pl.BlockSpec((BB,), lambda idx: (idx,)),
        out_shape=jax.ShapeDtypeStruct((B,), jnp.float32),
        compiler_params=pltpu.CompilerParams(
            dimension_semantics=("parallel",)),
    )(u_rows, i_rows,
      p, q, b1.reshape(1, d1),
      W2, b2.reshape(1, -1), W3, b3.reshape(1, -1), W4, b4.reshape(1, -1),
      wg.reshape(1, W), Wout[D:].reshape(1, -1), bout.reshape(1, 1))


def kernel(user_ids, item_ids, gmf_user, gmf_item, mlp_user, mlp_item,
           W1, b1, W2, b2, W3, b3, W4, b4, Wout, bout):
    tab_u, tab_i = _build_tables(gmf_user.T, mlp_user.T,
                                 gmf_item.T, mlp_item.T)
    u_rows, i_rows = _sc_gather2(user_ids, item_ids, tab_u, tab_i)
    return _tc_mlp(u_rows, i_rows, W1, b1, W2, b2, W3, b3, W4, b4, Wout, bout)


# speedup vs baseline: 2.1232x; 1.0548x over previous
import functools

import jax
import jax.numpy as jnp
from jax.experimental import pallas as pl
from jax.experimental.pallas import tpu as pltpu
from jax.experimental.pallas import tpu_sc as plsc


def _build_body(gu, mu, gi, mi, p1, p2, out_u, out_i):
    out_u[...] = jnp.concatenate(
        [jnp.swapaxes(gu[...], 0, 1), jnp.swapaxes(mu[...], 0, 1)], axis=1)
    out_i[...] = (
        jax.lax.dot_general(gi[...], p1[...], (((0,), (0,)), ((), ())),
                            preferred_element_type=jnp.float32)
        + jax.lax.dot_general(mi[...], p2[...], (((0,), (0,)), ((), ())),
                              preferred_element_type=jnp.float32))


def _build_tables(gu_t, mu_t, gi_t, mi_t):
    D, V = gu_t.shape
    BT = 8192
    grid = (pl.cdiv(V, BT),)
    inspec = pl.BlockSpec((D, BT), lambda i: (0, i))
    out_t = jax.ShapeDtypeStruct((V, 2 * D), jnp.float32)
    p1 = jnp.concatenate(
        [jnp.eye(D, dtype=jnp.float32), jnp.zeros((D, D), jnp.float32)],
        axis=1)
    p2 = jnp.concatenate(
        [jnp.zeros((D, D), jnp.float32), jnp.eye(D, dtype=jnp.float32)],
        axis=1)
    return pl.pallas_call(
        _build_body,
        grid=grid,
        in_specs=[inspec, inspec, inspec, inspec,
                  pl.BlockSpec((D, 2 * D), lambda i: (0, 0)),
                  pl.BlockSpec((D, 2 * D), lambda i: (0, 0))],
        out_specs=(pl.BlockSpec((BT, 2 * D), lambda i: (i, 0)),
                   pl.BlockSpec((BT, 2 * D), lambda i: (i, 0))),
        out_shape=(out_t, out_t),
        compiler_params=pltpu.CompilerParams(
            dimension_semantics=("arbitrary",)),
    )(gu_t, mu_t, gi_t, mi_t, p1, p2)


def _sc_gather2(user_ids, item_ids, tab_u, tab_i):
    B = user_ids.shape[0]
    W = tab_u.shape[1]
    info = plsc.get_sparse_core_info()
    nw = info.num_cores * info.num_subcores
    assert B % (8 * nw) == 0
    b_per_w = B // nw
    C = 256
    n_chunks = b_per_w // C
    mesh = plsc.VectorSubcoreMesh(core_axis_name="c", subcore_axis_name="s")
    out_t = jax.ShapeDtypeStruct((B, W), jnp.float32)

    @functools.partial(
        pl.kernel,
        mesh=mesh,
        out_type=(out_t, out_t),
        scratch_types=[
            pltpu.VMEM((C,), jnp.int32),
            pltpu.VMEM((C,), jnp.int32),
            pltpu.VMEM((C, W), jnp.float32),
            pltpu.VMEM((C, W), jnp.float32),
            pltpu.SemaphoreType.DMA,
            pltpu.SemaphoreType.DMA,
        ],
    )
    def k(uid_hbm, iid_hbm, tu, ti, o_u, o_i,
          idx_u, idx_i, rows_u, rows_i, sem_u, sem_i):
        wid = jax.lax.axis_index("s") * info.num_cores + jax.lax.axis_index("c")
        for c in range(n_chunks):
            base = wid * b_per_w + c * C
            sl = pl.ds(base, C)
            pltpu.sync_copy(uid_hbm.at[sl], idx_u)
            pltpu.sync_copy(iid_hbm.at[sl], idx_i)
            cp_u = pltpu.async_copy(tu.at[idx_u], rows_u, sem_u)
            cp_i = pltpu.async_copy(ti.at[idx_i], rows_i, sem_i)
            cp_u.wait()
            pltpu.sync_copy(rows_u, o_u.at[sl])
            cp_i.wait()
            pltpu.sync_copy(rows_i, o_i.at[sl])

    return k(user_ids, item_ids, tab_u, tab_i)


def _tc_body(u, i, p, q, b1, w2, b2, w3, b3, w4, b4, wg, wx, bout, out):
    uv = u[...]
    iv = i[...]
    h = jnp.maximum(
        jnp.dot(uv, p[...], preferred_element_type=jnp.float32)
        + jnp.dot(iv, q[...], preferred_element_type=jnp.float32)
        + b1[...], 0.0)
    h = jnp.maximum(
        jnp.dot(h, w2[...], preferred_element_type=jnp.float32) + b2[...], 0.0)
    h = jnp.maximum(
        jnp.dot(h, w3[...], preferred_element_type=jnp.float32) + b3[...], 0.0)
    h = jnp.maximum(
        jnp.dot(h, w4[...], preferred_element_type=jnp.float32) + b4[...], 0.0)
    pred = (jnp.sum(uv * iv * wg[...], axis=1)
            + jnp.sum(h * wx[...], axis=1) + bout[0, 0])
    out[...] = jax.nn.sigmoid(pred)


def _tc_mlp(u_rows, i_rows, W1, b1, W2, b2, W3, b3, W4, b4, Wout, bout):
    B, W = u_rows.shape
    D = W // 2
    BB = 2048
    grid = (B // BB,)
    d1 = W1.shape[1]
    zpad = jnp.zeros((D, d1), jnp.float32)
    p = jnp.concatenate([zpad, W1[:D]], axis=0)
    q = jnp.concatenate([zpad, W1[D:]], axis=0)
    wg = jnp.concatenate([Wout[:D, 0], jnp.zeros((D,), jnp.float32)])
    row = lambda m, n: pl.BlockSpec((m, n), lambda idx: (0, 0))
    blk = lambda n: pl.BlockSpec((BB, n), lambda idx: (idx, 0))
    return pl.pallas_call(
        _tc_body,
        grid=grid,
        in_specs=[
            blk(W), blk(W),
            row(W, d1), row(W, d1), row(1, d1),
            row(d1, W2.shape[1]), row(1, W2.shape[1]),
            row(W3.shape[0], W3.shape[1]), row(1, W3.shape[1]),
            row(W4.shape[0], W4.shape[1]), row(1, W4.shape[1]),
            row(1, W), row(1, W4.shape[1]), row(1, 1),
        ],
        out_specs=pl.BlockSpec((BB,), lambda idx: (idx,)),
        out_shape=jax.ShapeDtypeStruct((B,), jnp.float32),
        compiler_params=pltpu.CompilerParams(
            dimension_semantics=("parallel",)),
    )(u_rows, i_rows,
      p, q, b1.reshape(1, d1),
      W2, b2.reshape(1, -1), W3, b3.reshape(1, -1), W4, b4.reshape(1, -1),
      wg.reshape(1, W), Wout[D:].reshape(1, -1), bout.reshape(1, 1))


def kernel(user_ids, item_ids, gmf_user, gmf_item, mlp_user, mlp_item,
           W1, b1, W2, b2, W3, b3, W4, b4, Wout, bout):
    tab_u, tab_i = _build_tables(gmf_user.T, mlp_user.T,
                                 gmf_item.T, mlp_item.T)
    u_rows, i_rows = _sc_gather2(user_ids, item_ids, tab_u, tab_i)
    return _tc_mlp(u_rows, i_rows, W1, b1, W2, b2, W3, b3, W4, b4, Wout, bout)
